# baseline (speedup 1.0000x reference)
import functools

import jax
import jax.numpy as jnp
from jax import lax
from jax.experimental import pallas as pl
from jax.experimental.pallas import tpu as pltpu
from jax.experimental.pallas import tpu_sc as plsc

B_ROWS = 16384
SEQ = 200
D = 64
WIDE = 2 * D
LANES = 16
TOTAL = B_ROWS * SEQ
IDX_MINOR = 128
ROWS = TOTAL // IDX_MINOR
NUM_WORKERS = 32
ROWS_PER_W = ROWS // NUM_WORKERS
STEPS = ROWS_PER_W
NBUF = 4


def _make_gather():
    mesh = plsc.VectorSubcoreMesh(core_axis_name="c", subcore_axis_name="s")

    @functools.partial(
        pl.kernel,
        mesh=mesh,
        out_type=jax.ShapeDtypeStruct((ROWS, IDX_MINOR, D), jnp.float32),
        scratch_types=[
            pltpu.VMEM((NBUF, IDX_MINOR), jnp.int32),
            pltpu.VMEM((NBUF, IDX_MINOR, WIDE), jnp.float32),
            pltpu.VMEM((NBUF, IDX_MINOR, D), jnp.float32),
            pltpu.SemaphoreType.DMA((NBUF,)),
            pltpu.SemaphoreType.DMA((NBUF,)),
            pltpu.SemaphoreType.DMA((NBUF,)),
        ],
        compiler_params=pltpu.CompilerParams(use_tc_tiling_on_sc=False),
    )
    def gather_kernel(idx_hbm, table_hbm, out_hbm, idx_v, rows_v, cmp_v,
                      sem_i, sem_g, sem_o):
        wid = lax.axis_index("s") * 2 + lax.axis_index("c")
        base = wid * ROWS_PER_W

        def idx_cp(step, b):
            return pltpu.make_async_copy(
                idx_hbm.at[pl.ds(base + step, 1)],
                idx_v.at[pl.ds(b, 1)], sem_i.at[b])

        def gather_cp(b):
            return pltpu.make_async_copy(
                table_hbm.at[idx_v.at[b]], rows_v.at[b], sem_g.at[b])

        def store_cp(step, b):
            return pltpu.make_async_copy(
                cmp_v.at[b], out_hbm.at[base + step], sem_o.at[b])

        def compact(b):
            @plsc.parallel_loop(0, IDX_MINOR, 2, unroll=8)
            def _(r):
                vals = [rows_v[b, r + p, pl.ds(LANES * q, LANES)]
                        for p in range(2) for q in range(D // LANES)]
                for p in range(2):
                    for q in range(D // LANES):
                        cmp_v[b, r + p, pl.ds(LANES * q, LANES)] = (
                            vals[p * (D // LANES) + q])

        for b in range(NBUF):
            idx_cp(b, b).start()
        for b in range(NBUF):
            idx_cp(b, b).wait()
            gather_cp(b).start()

        def body(i, carry):
            for b in range(NBUF):
                s = NBUF * i + b
                gather_cp(b).wait()

                @pl.when(s + NBUF < STEPS)
                def _():
                    idx_cp(s + NBUF, b).start()

                @pl.when(i > 0)
                def _():
                    store_cp(s - NBUF, b).wait()

                compact(b)
                store_cp(s, b).start()

                @pl.when(s + NBUF < STEPS)
                def _():
                    idx_cp(s + NBUF, b).wait()
                    gather_cp(b).start()

            return carry

        lax.fori_loop(0, STEPS // NBUF, body, 0)

        for b in range(NBUF):
            store_cp(STEPS - NBUF + b, b).wait()

    return gather_kernel


_gather = _make_gather()


def kernel(idxes, pe):
    idx2 = idxes.reshape(ROWS, IDX_MINOR)
    table2 = jnp.concatenate([pe, pe], axis=1)
    out = _gather(idx2, table2)
    return out.reshape(B_ROWS, SEQ, D)


# speedup vs baseline: 1.1146x; 1.1146x over previous
import functools

import jax
import jax.numpy as jnp
from jax import lax
from jax.experimental import pallas as pl
from jax.experimental.pallas import tpu as pltpu
from jax.experimental.pallas import tpu_sc as plsc

B_ROWS = 16384
SEQ = 200
D = 64
TOTAL = B_ROWS * SEQ
IDX_MINOR = 128
ROWS = TOTAL // IDX_MINOR
NUM_WORKERS = 32
ROWS_PER_W = ROWS // NUM_WORKERS
S = 2
STEPS = ROWS_PER_W // S
NBUF = 5


def _make_gather():
    mesh = plsc.VectorSubcoreMesh(core_axis_name="c", subcore_axis_name="s")

    @functools.partial(
        pl.kernel,
        mesh=mesh,
        out_type=jax.ShapeDtypeStruct((ROWS, IDX_MINOR, D), jnp.float32),
        scratch_types=[
            pltpu.VMEM((NBUF, S, IDX_MINOR), jnp.int32),
            pltpu.VMEM((NBUF, S, IDX_MINOR, D), jnp.float32),
            pltpu.SemaphoreType.DMA((NBUF,)),
            pltpu.SemaphoreType.DMA((NBUF,)),
            pltpu.SemaphoreType.DMA((NBUF,)),
        ],
        compiler_params=pltpu.CompilerParams(use_tc_tiling_on_sc=False),
    )
    def gather_kernel(idx_hbm, table_hbm, out_hbm, idx_v, rows_v,
                      sem_i, sem_g, sem_o):
        wid = lax.axis_index("s") * 2 + lax.axis_index("c")
        base = wid * ROWS_PER_W

        def idx_cp(step, b):
            return pltpu.make_async_copy(
                idx_hbm.at[pl.ds(base + step * S, S)], idx_v.at[b], sem_i.at[b])

        def gather_cp(b, j):
            return pltpu.make_async_copy(
                table_hbm.at[idx_v.at[b].at[j]], rows_v.at[b].at[j],
                sem_g.at[b])

        def store_cp(step, b):
            return pltpu.make_async_copy(
                rows_v.at[b], out_hbm.at[pl.ds(base + step * S, S)], sem_o.at[b])

        for b in range(NBUF):
            idx_cp(b, b).start()

        def body(i, carry):
            for b in range(NBUF):
                s = NBUF * i + b
                idx_cp(s, b).wait()

                @pl.when(i > 0)
                def _():
                    store_cp(s - NBUF, b).wait()

                for j in range(S):
                    gather_cp(b, j).start()
            for b in range(NBUF):
                s = NBUF * i + b
                for j in range(S):
                    gather_cp(b, j).wait()
                store_cp(s, b).start()

                @pl.when(s + NBUF < STEPS)
                def _():
                    idx_cp(s + NBUF, b).start()

            return carry

        lax.fori_loop(0, STEPS // NBUF, body, 0)

        for b in range(NBUF):
            store_cp(STEPS - NBUF + b, b).wait()

    return gather_kernel


_gather = _make_gather()


def kernel(idxes, pe):
    idx2 = idxes.reshape(ROWS, IDX_MINOR)
    out = _gather(idx2, pe)
    return out.reshape(B_ROWS, SEQ, D)
